# baseline (speedup 1.0000x reference)
import jax
import jax.numpy as jnp
from jax import lax
from jax.experimental import pallas as pl
from jax.experimental.pallas import tpu as pltpu
from jax.experimental.pallas import tpu_sc as plsc

NC, NS = 2, 16
NW = NC * NS
LANES = 16


def _sc_loss_partials(img_tiled, ia, ib, P, wpb):
    own = P // wpb
    C = -(-own // 128) * 128
    for s in range(wpb):
        smod = s * own
        pre_min = max(smod + C - P, 0)
        pre = pre_min + ((smod - pre_min) & 7)
        assert 0 <= smod - pre and smod - pre + C <= P and (smod - pre) % 8 == 0
        assert pre + own <= C
    assert wpb & (wpb - 1) == 0
    wpb_shift = wpb.bit_length() - 1

    mesh = plsc.VectorSubcoreMesh(core_axis_name="c", subcore_axis_name="s")

    NQ = 4
    QR = C // (128 * NQ)
    QC = QR * 128

    def body(img, ia_h, ib_h, out_h,
             ia_v, ib_v, iac_v, za_v, zb_v, acc_v, ssem, *qsems):
        cc = lax.axis_index("c")
        ss = lax.axis_index("s")
        wid = ss * NC + cc
        b = wid >> wpb_shift
        slot = wid - (b << wpb_shift)
        smod = slot * own
        pre_min = lax.max(smod + (C - P), 0)
        pre = pre_min + ((smod - pre_min) & 7)
        fstart = pl.multiple_of(b * P + smod - pre, 8)

        da_s = pltpu.async_copy(ia_h.at[pl.ds(fstart, C)], ia_v, ssem)
        db_s = pltpu.async_copy(ib_h.at[pl.ds(fstart, C)], ib_v, ssem)
        da_s.wait()

        def strip(j, carry):
            for k in range(8):
                sl = pl.ds(j * 128 + k * 16, 16)
                iac_v[sl] = ia_v[sl] & 0xFFFFF
            return carry

        lax.fori_loop(0, C // 128, strip, 0)
        db_s.wait()
        gathers = []
        for qi in range(NQ):
            qs = pl.ds(qi * QC, QC)
            gathers.append((
                pltpu.async_copy(img.at[iac_v.at[qs]], za_v.at[qs],
                                 qsems[qi]),
                pltpu.async_copy(img.at[ib_v.at[qs]], zb_v.at[qs],
                                 qsems[qi]),
            ))

        lane = lax.iota(jnp.int32, LANES)
        lo = pre
        hi = pre + own

        def loss_rows(j, acc):
            base = j * 128
            for k in range(8):
                off = base + k * 16
                sl = pl.ds(off, 16)
                gt = (ia_v[sl] >> 20).astype(jnp.float32) - 1.0
                diff = za_v[sl] - zb_v[sl]
                q = gt * diff
                u = jnp.exp(-jnp.abs(q))
                t = u / (2.0 + u)
                t2 = t * t
                sp = t * (2.0 + t2 * (2.0 / 3.0 + t2 * (
                    2.0 / 5.0 + t2 * (2.0 / 7.0 + t2 * (2.0 / 9.0)))))
                sp = jnp.maximum(-q, 0.0) + sp
                m = jnp.abs(gt)
                lv = m * sp + (1.0 - m) * (diff * diff)
                li = lane + off
                sel = (li >= lo) & (li < hi)
                acc = acc + jnp.where(sel, lv, 0.0)
            return acc

        acc = jnp.zeros((LANES,), jnp.float32)
        for qi in range(NQ):
            ga, gb = gathers[qi]
            ga.wait()
            gb.wait()
            acc = lax.fori_loop(qi * QR, (qi + 1) * QR, loss_rows, acc)
        acc_v[...] = acc
        pltpu.sync_copy(acc_v, out_h.at[pl.ds(wid * LANES, LANES)])

    f = pl.kernel(
        body,
        out_type=jax.ShapeDtypeStruct((NW * LANES,), jnp.float32),
        mesh=mesh,
        scratch_types=[
            pltpu.VMEM((C,), jnp.int32),
            pltpu.VMEM((C,), jnp.int32),
            pltpu.VMEM((C,), jnp.int32),
            pltpu.VMEM((C,), jnp.float32),
            pltpu.VMEM((C,), jnp.float32),
            pltpu.VMEM((LANES,), jnp.float32),
            pltpu.SemaphoreType.DMA,
        ] + [pltpu.SemaphoreType.DMA] * NQ,
    )
    return f(img_tiled, ia, ib)


def _tc_reduce(partials, n_total):
    def body(p_ref, out_ref):
        out_ref[0, 0] = jnp.sum(p_ref[...]) / n_total

    return pl.pallas_call(
        body,
        out_shape=jax.ShapeDtypeStruct((1, 1), jnp.float32),
        out_specs=pl.BlockSpec(memory_space=pltpu.SMEM),
    )(partials)


def kernel(input, x_A, y_A, x_B, y_B, ordinal):
    B, _, H, W = input.shape
    P = x_A.shape[1]
    wpb = NW // B
    HW = H * W
    XT = W // 128

    timg = (input.reshape(B, H // 8, 8, XT, 128)
            .transpose(0, 1, 3, 2, 4).reshape(-1))

    boffs = (jnp.arange(B, dtype=jnp.int32) * HW)[:, None]

    def addr(y, x):
        y = y.astype(jnp.int32)
        x = x.astype(jnp.int32)
        return (boffs + (y >> 3) * (XT * 1024) + (x >> 7) * 1024
                + (y & 7) * 128 + (x & 127)).reshape(-1)

    assert B * HW <= 1 << 20
    ia = addr(y_A, x_A) | (ordinal.astype(jnp.int32) << 20).reshape(-1)
    ib = addr(y_B, x_B)

    partials = _sc_loss_partials(timg, ia, ib, P, wpb)

    loss = _tc_reduce(partials.reshape(4, NW * LANES // 4), B * P)
    return loss.reshape(1)


# speedup vs baseline: 1.0079x; 1.0079x over previous
import jax
import jax.numpy as jnp
from jax import lax
from jax.experimental import pallas as pl
from jax.experimental.pallas import tpu as pltpu
from jax.experimental.pallas import tpu_sc as plsc

NC, NS = 2, 16
NW = NC * NS
LANES = 16


def _sc_loss_partials(img_tiled, ia, ib, P, wpb):
    own = P // wpb
    C = -(-own // 128) * 128
    for s in range(wpb):
        smod = s * own
        pre_min = max(smod + C - P, 0)
        pre = pre_min + ((smod - pre_min) & 7)
        assert 0 <= smod - pre and smod - pre + C <= P and (smod - pre) % 8 == 0
        assert pre + own <= C
    assert wpb & (wpb - 1) == 0
    wpb_shift = wpb.bit_length() - 1

    mesh = plsc.VectorSubcoreMesh(core_axis_name="c", subcore_axis_name="s")

    HR = C // 256
    HC = HR * 128

    def body(img, ia_h, ib_h, out_h,
             ia_v, ib_v, iac_v, za_v, zb_v, acc_v, ssem, g0sem, g1sem):
        cc = lax.axis_index("c")
        ss = lax.axis_index("s")
        wid = ss * NC + cc
        b = wid >> wpb_shift
        slot = wid - (b << wpb_shift)
        smod = slot * own
        pre_min = lax.max(smod + (C - P), 0)
        pre = pre_min + ((smod - pre_min) & 7)
        fstart = pl.multiple_of(b * P + smod - pre, 8)

        da_s = pltpu.async_copy(ia_h.at[pl.ds(fstart, C)], ia_v, ssem)
        db_s = pltpu.async_copy(ib_h.at[pl.ds(fstart, C)], ib_v, ssem)
        da_s.wait()

        def strip(j, carry):
            for k in range(8):
                sl = pl.ds(j * 128 + k * 16, 16)
                iac_v[sl] = ia_v[sl] & 0xFFFFF
            return carry

        lax.fori_loop(0, HR, strip, 0)
        ga0 = pltpu.async_copy(img.at[iac_v.at[pl.ds(0, HC)]],
                               za_v.at[pl.ds(0, HC)], g0sem)
        db_s.wait()
        gb0 = pltpu.async_copy(img.at[ib_v.at[pl.ds(0, HC)]],
                               zb_v.at[pl.ds(0, HC)], g0sem)
        lax.fori_loop(HR, 2 * HR, strip, 0)
        ga1 = pltpu.async_copy(img.at[iac_v.at[pl.ds(HC, HC)]],
                               za_v.at[pl.ds(HC, HC)], g1sem)
        gb1 = pltpu.async_copy(img.at[ib_v.at[pl.ds(HC, HC)]],
                               zb_v.at[pl.ds(HC, HC)], g1sem)

        lane = lax.iota(jnp.int32, LANES)
        lo = pre
        hi = pre + own

        def loss_rows(j, acc):
            base = j * 128
            for k in range(8):
                off = base + k * 16
                sl = pl.ds(off, 16)
                gt = (ia_v[sl] >> 20).astype(jnp.float32) - 1.0
                diff = za_v[sl] - zb_v[sl]
                q = gt * diff
                u = jnp.exp(-jnp.abs(q))
                t = u / (2.0 + u)
                t2 = t * t
                sp = t * (2.0 + t2 * (2.0 / 3.0 + t2 * (
                    2.0 / 5.0 + t2 * (2.0 / 7.0 + t2 * (2.0 / 9.0)))))
                sp = jnp.maximum(-q, 0.0) + sp
                m = jnp.abs(gt)
                lv = m * sp + (1.0 - m) * (diff * diff)
                li = lane + off
                sel = (li >= lo) & (li < hi)
                acc = acc + jnp.where(sel, lv, 0.0)
            return acc

        ga0.wait()
        gb0.wait()
        acc = lax.fori_loop(0, HR, loss_rows,
                            jnp.zeros((LANES,), jnp.float32))
        ga1.wait()
        gb1.wait()
        acc = lax.fori_loop(HR, 2 * HR, loss_rows, acc)
        acc_v[...] = acc
        pltpu.sync_copy(acc_v, out_h.at[pl.ds(wid * LANES, LANES)])

    f = pl.kernel(
        body,
        out_type=jax.ShapeDtypeStruct((NW * LANES,), jnp.float32),
        mesh=mesh,
        scratch_types=[
            pltpu.VMEM((C,), jnp.int32),
            pltpu.VMEM((C,), jnp.int32),
            pltpu.VMEM((C,), jnp.int32),
            pltpu.VMEM((C,), jnp.float32),
            pltpu.VMEM((C,), jnp.float32),
            pltpu.VMEM((LANES,), jnp.float32),
            pltpu.SemaphoreType.DMA,
            pltpu.SemaphoreType.DMA,
            pltpu.SemaphoreType.DMA,
        ],
        compiler_params=pltpu.CompilerParams(skip_device_barrier=True),
    )
    return f(img_tiled, ia, ib)


def _tc_reduce(partials, n_total):
    def body(p_ref, out_ref):
        out_ref[0, 0] = jnp.sum(p_ref[...]) / n_total

    return pl.pallas_call(
        body,
        out_shape=jax.ShapeDtypeStruct((1, 1), jnp.float32),
        out_specs=pl.BlockSpec(memory_space=pltpu.SMEM),
    )(partials)


def kernel(input, x_A, y_A, x_B, y_B, ordinal):
    B, _, H, W = input.shape
    P = x_A.shape[1]
    wpb = NW // B
    HW = H * W
    XT = W // 128

    timg = (input.reshape(B, H // 8, 8, XT, 128)
            .transpose(0, 1, 3, 2, 4).reshape(-1))

    boffs = (jnp.arange(B, dtype=jnp.int32) * HW)[:, None]

    def addr(y, x):
        y = y.astype(jnp.int32)
        x = x.astype(jnp.int32)
        return (boffs + (y >> 3) * (XT * 1024) + (x >> 7) * 1024
                + (y & 7) * 128 + (x & 127)).reshape(-1)

    assert B * HW <= 1 << 20
    ia = addr(y_A, x_A) | (ordinal.astype(jnp.int32) << 20).reshape(-1)
    ib = addr(y_B, x_B)

    partials = _sc_loss_partials(timg, ia, ib, P, wpb)

    loss = _tc_reduce(partials.reshape(4, NW * LANES // 4), B * P)
    return loss.reshape(1)


# speedup vs baseline: 1.0114x; 1.0035x over previous
import jax
import jax.numpy as jnp
from jax import lax
from jax.experimental import pallas as pl
from jax.experimental.pallas import tpu as pltpu
from jax.experimental.pallas import tpu_sc as plsc

NC, NS = 2, 16
NW = NC * NS
LANES = 16


def _sc_loss_partials(img_tiled, ia, ib, P, wpb):
    own = P // wpb
    C = -(-own // 128) * 128
    for s in range(wpb):
        smod = s * own
        pre_min = max(smod + C - P, 0)
        pre = pre_min + ((smod - pre_min) & 7)
        assert 0 <= smod - pre and smod - pre + C <= P and (smod - pre) % 8 == 0
        assert pre + own <= C
    assert wpb & (wpb - 1) == 0
    wpb_shift = wpb.bit_length() - 1

    mesh = plsc.VectorSubcoreMesh(core_axis_name="c", subcore_axis_name="s")

    HR = C // 256
    HC = HR * 128

    def body(img, ia_h, ib_h, out_h,
             ia_v, ib_v, iac_v, za_v, zb_v, acc_v, ssem, g0sem, g1sem):
        cc = lax.axis_index("c")
        ss = lax.axis_index("s")
        wid = ss * NC + cc
        b = wid >> wpb_shift
        slot = wid - (b << wpb_shift)
        smod = slot * own
        pre_min = lax.max(smod + (C - P), 0)
        pre = pre_min + ((smod - pre_min) & 7)
        fstart = pl.multiple_of(b * P + smod - pre, 8)

        da_s = pltpu.async_copy(ia_h.at[pl.ds(fstart, C)], ia_v, ssem)
        db_s = pltpu.async_copy(ib_h.at[pl.ds(fstart, C)], ib_v, ssem)
        da_s.wait()

        def strip(j, carry):
            for k in range(8):
                sl = pl.ds(j * 128 + k * 16, 16)
                iac_v[sl] = ia_v[sl] & 0xFFFFF
            return carry

        lax.fori_loop(0, HR, strip, 0)
        ga0 = pltpu.async_copy(img.at[iac_v.at[pl.ds(0, HC)]],
                               za_v.at[pl.ds(0, HC)], g0sem)
        db_s.wait()
        gb0 = pltpu.async_copy(img.at[ib_v.at[pl.ds(0, HC)]],
                               zb_v.at[pl.ds(0, HC)], g0sem)
        lax.fori_loop(HR, 2 * HR, strip, 0)
        ga1 = pltpu.async_copy(img.at[iac_v.at[pl.ds(HC, HC)]],
                               za_v.at[pl.ds(HC, HC)], g1sem)
        gb1 = pltpu.async_copy(img.at[ib_v.at[pl.ds(HC, HC)]],
                               zb_v.at[pl.ds(HC, HC)], g1sem)

        lane = lax.iota(jnp.int32, LANES)
        lo = pre
        hi = pre + own

        def loss_rows(j, acc):
            base = j * 128
            for k in range(8):
                off = base + k * 16
                sl = pl.ds(off, 16)
                gt = (ia_v[sl] >> 20).astype(jnp.float32) - 1.0
                diff = za_v[sl] - zb_v[sl]
                q = gt * diff
                u = jnp.exp(-jnp.abs(q))
                t = u / (2.0 + u)
                t2 = t * t
                sp = t * (2.0 + t2 * (2.0 / 3.0 + t2 * (
                    2.0 / 5.0 + t2 * (2.0 / 7.0 + t2 * (2.0 / 9.0)))))
                sp = jnp.maximum(-q, 0.0) + sp
                m = jnp.abs(gt)
                lv = m * sp + (1.0 - m) * (diff * diff)
                li = lane + off
                sel = (li >= lo) & (li < hi)
                acc = acc + jnp.where(sel, lv, 0.0)
            return acc

        ga0.wait()
        gb0.wait()
        acc = lax.fori_loop(0, HR, loss_rows,
                            jnp.zeros((LANES,), jnp.float32))
        ga1.wait()
        gb1.wait()
        acc = lax.fori_loop(HR, 2 * HR, loss_rows, acc)
        acc_v[...] = acc
        pltpu.sync_copy(acc_v, out_h.at[pl.ds(wid * LANES, LANES)])

    f = pl.kernel(
        body,
        out_type=jax.ShapeDtypeStruct((NW * LANES,), jnp.float32),
        mesh=mesh,
        scratch_types=[
            pltpu.VMEM((C,), jnp.int32),
            pltpu.VMEM((C,), jnp.int32),
            pltpu.VMEM((C,), jnp.int32),
            pltpu.VMEM((C,), jnp.float32),
            pltpu.VMEM((C,), jnp.float32),
            pltpu.VMEM((LANES,), jnp.float32),
            pltpu.SemaphoreType.DMA,
            pltpu.SemaphoreType.DMA,
            pltpu.SemaphoreType.DMA,
        ],
    )
    return f(img_tiled, ia, ib)


def _tc_reduce(partials, n_total):
    def body(p_ref, out_ref):
        out_ref[0, 0] = jnp.sum(p_ref[...]) / n_total

    return pl.pallas_call(
        body,
        out_shape=jax.ShapeDtypeStruct((1, 1), jnp.float32),
        out_specs=pl.BlockSpec(memory_space=pltpu.SMEM),
    )(partials)


def kernel(input, x_A, y_A, x_B, y_B, ordinal):
    B, _, H, W = input.shape
    P = x_A.shape[1]
    wpb = NW // B
    HW = H * W
    XT = W // 128

    timg = (input.reshape(B, H // 8, 8, XT, 128)
            .transpose(0, 1, 3, 2, 4).reshape(-1))

    boffs = (jnp.arange(B, dtype=jnp.int32) * HW)[:, None]

    def addr(y, x):
        y = y.astype(jnp.int32)
        x = x.astype(jnp.int32)
        return (boffs + (y >> 3) * (XT * 1024) + (x >> 7) * 1024
                + (y & 7) * 128 + (x & 127)).reshape(-1)

    assert B * HW <= 1 << 20
    ia = addr(y_A, x_A) | (ordinal.astype(jnp.int32) << 20).reshape(-1)
    ib = addr(y_B, x_B)

    partials = _sc_loss_partials(timg, ia, ib, P, wpb)

    loss = _tc_reduce(partials.reshape(4, NW * LANES // 4), B * P)
    return loss.reshape(1)
